# baseline (speedup 1.0000x reference)
import functools

import jax
import jax.numpy as jnp
from jax import lax
from jax.experimental import pallas as pl
from jax.experimental.pallas import tpu as pltpu


def _fused_kernel(adj_ref, x_ref, w1_ref, b1_ref, w2_ref, b2_ref,
                  o_ref, s1_ref, s2_ref, *, nm, bm):
    i = pl.program_id(0)

    @pl.when(i == 0)
    def _prologue():
        s1_ref[...] = jnp.dot(x_ref[...], w1_ref[...],
                              preferred_element_type=jnp.float32)

    @pl.when(i < nm)
    def _phase1():
        acc = jnp.dot(adj_ref[...], s1_ref[...],
                      preferred_element_type=jnp.float32)
        h = jnp.maximum(acc + b1_ref[...], 0.0)
        s2_ref[pl.ds(i * bm, bm), :] = jnp.dot(
            h, w2_ref[...], preferred_element_type=jnp.float32)

    @pl.when(i >= nm)
    def _phase2():
        o = jnp.dot(adj_ref[...], s2_ref[...],
                    preferred_element_type=jnp.float32) + b2_ref[...]
        m = jnp.max(o, axis=1, keepdims=True)
        e = o - m
        lse = jnp.log(jnp.sum(jnp.exp(e), axis=1, keepdims=True))
        o_ref[...] = e - lse


def kernel(x, adj, W1, b1, W2, b2):
    n, nfeat = x.shape
    nhid = W1.shape[1]
    nclass = W2.shape[1]

    bm = 200 if n % 400 == 0 else n
    nm = n // bm

    out = pl.pallas_call(
        functools.partial(_fused_kernel, nm=nm, bm=bm),
        grid=(2 * nm,),
        in_specs=[
            pl.BlockSpec((bm, n), lambda i: (lax.rem(i, nm), 0)),
            pl.BlockSpec((n, nfeat), lambda i: (0, 0)),
            pl.BlockSpec((nfeat, nhid), lambda i: (0, 0)),
            pl.BlockSpec((1, nhid), lambda i: (0, 0)),
            pl.BlockSpec((nhid, nclass), lambda i: (0, 0)),
            pl.BlockSpec((1, nclass), lambda i: (0, 0)),
        ],
        out_specs=pl.BlockSpec(
            (bm, nclass), lambda i: (jnp.maximum(i - nm, 0), 0)),
        out_shape=jax.ShapeDtypeStruct((n, nclass), jnp.float32),
        scratch_shapes=[
            pltpu.VMEM((n, nhid), jnp.float32),
            pltpu.VMEM((n, nclass), jnp.float32),
        ],
        compiler_params=pltpu.CompilerParams(
            dimension_semantics=("arbitrary",)),
    )(adj, x, W1, b1.reshape(1, nhid), W2, b2.reshape(1, nclass))

    return out


# speedup vs baseline: 1.0617x; 1.0617x over previous
import functools

import jax
import jax.numpy as jnp
from jax.experimental import pallas as pl
from jax.experimental.pallas import tpu as pltpu


def _log_softmax(o):
    m = jnp.max(o, axis=1, keepdims=True)
    e = o - m
    return e - jnp.log(jnp.sum(jnp.exp(e), axis=1, keepdims=True))


def _fused_kernel(adj_ref, x_ref, w1_ref, b1_ref, w2_ref, b2_ref,
                  om_ref, oc_ref, s1_ref, s2_ref, cache_ref,
                  *, nm, bm, nc):
    i = pl.program_id(0)

    @pl.when(i == 0)
    def _prologue():
        s1_ref[...] = jnp.dot(x_ref[...], w1_ref[...],
                              preferred_element_type=jnp.float32)

    @pl.when(i < nm)
    def _phase1():
        acc = jnp.dot(adj_ref[...], s1_ref[...],
                      preferred_element_type=jnp.float32)
        h = jnp.maximum(acc + b1_ref[...], 0.0)
        s2_ref[pl.ds(i * bm, bm), :] = jnp.dot(
            h, w2_ref[...], preferred_element_type=jnp.float32)

    @pl.when((i >= nm - nc) & (i < nm))
    def _fill_cache():
        cache_ref[pl.ds((i - (nm - nc)) * bm, bm), :] = (
            adj_ref[...].astype(jnp.bfloat16))

    @pl.when(i >= nm)
    def _phase2_stream():
        o = jnp.dot(adj_ref[...], s2_ref[...],
                    preferred_element_type=jnp.float32) + b2_ref[...]
        om_ref[...] = _log_softmax(o)

    @pl.when((i >= nm) & (i < nm + nc))
    def _phase2_cached():
        ab = cache_ref[pl.ds((i - nm) * bm, bm), :]
        o = jnp.dot(ab, s2_ref[...].astype(jnp.bfloat16),
                    preferred_element_type=jnp.float32) + b2_ref[...]
        oc_ref[...] = _log_softmax(o)


def kernel(x, adj, W1, b1, W2, b2):
    n, nfeat = x.shape
    nhid = W1.shape[1]
    nclass = W2.shape[1]

    if n % 400 == 0:
        bm, nc = 400, 2
    else:
        bm, nc = n // 2, 1
    nm = n // bm

    def adj_idx(i):
        return (jnp.where(i < nm, i, i - nm), 0)

    def om_idx(i):
        return (jnp.maximum(i - nm, 0), 0)

    def oc_idx(i):
        return (jnp.clip(i - nm, 0, nc - 1), 0)

    out_main, out_cached = pl.pallas_call(
        functools.partial(_fused_kernel, nm=nm, bm=bm, nc=nc),
        grid=(2 * nm - nc,),
        in_specs=[
            pl.BlockSpec((bm, n), adj_idx),
            pl.BlockSpec((n, nfeat), lambda i: (0, 0)),
            pl.BlockSpec((nfeat, nhid), lambda i: (0, 0)),
            pl.BlockSpec((1, nhid), lambda i: (0, 0)),
            pl.BlockSpec((nhid, nclass), lambda i: (0, 0)),
            pl.BlockSpec((1, nclass), lambda i: (0, 0)),
        ],
        out_specs=[
            pl.BlockSpec((bm, nclass), om_idx),
            pl.BlockSpec((bm, nclass), oc_idx),
        ],
        out_shape=[
            jax.ShapeDtypeStruct(((nm - nc) * bm, nclass), jnp.float32),
            jax.ShapeDtypeStruct((nc * bm, nclass), jnp.float32),
        ],
        scratch_shapes=[
            pltpu.VMEM((n, nhid), jnp.float32),
            pltpu.VMEM((n, nclass), jnp.float32),
            pltpu.VMEM((nc * bm, n), jnp.bfloat16),
        ],
        compiler_params=pltpu.CompilerParams(
            dimension_semantics=("arbitrary",),
            vmem_limit_bytes=112 * 1024 * 1024),
    )(adj, x, W1, b1.reshape(1, nhid), W2, b2.reshape(1, nclass))

    return jnp.concatenate([out_main, out_cached], axis=0)
